# baseline (speedup 1.0000x reference)
import functools

import jax
import jax.numpy as jnp
from jax import lax
from jax.experimental import pallas as pl
from jax.experimental.pallas import tpu as pltpu
from jax.experimental.pallas import tpu_sc as plsc

EMB_K = 64
IDX_W = 128


def _make_gather_kernel(batch, emb_k, n_workers):
    rows_total = batch // IDX_W
    rows_per_w = rows_total // n_workers
    mesh = plsc.VectorSubcoreMesh(core_axis_name="c", subcore_axis_name="s")

    @functools.partial(
        pl.kernel,
        mesh=mesh,
        compiler_params=pltpu.CompilerParams(use_tc_tiling_on_sc=False),
        out_type=[
            jax.ShapeDtypeStruct((rows_total, IDX_W, emb_k), jnp.float32),
            jax.ShapeDtypeStruct((rows_total, IDX_W, emb_k), jnp.float32),
        ],
        scratch_types=[
            pltpu.VMEM((rows_per_w, IDX_W), jnp.int32),
            pltpu.VMEM((rows_per_w, IDX_W), jnp.int32),
            pltpu.VMEM((rows_per_w, IDX_W, emb_k), jnp.float32),
            pltpu.VMEM((rows_per_w, IDX_W, emb_k), jnp.float32),
            pltpu.SemaphoreType.DMA,
        ],
    )
    def gather_kernel(uidx_hbm, iidx_hbm, utab_hbm, itab_hbm,
                      uout_hbm, iout_hbm,
                      uidx_v, iidx_v, urows_v, irows_v, sem):
        wid = lax.axis_index("s") * 2 + lax.axis_index("c")
        base = wid * rows_per_w
        pltpu.sync_copy(uidx_hbm.at[pl.ds(base, rows_per_w)], uidx_v)
        pltpu.sync_copy(iidx_hbm.at[pl.ds(base, rows_per_w)], iidx_v)
        copies = []
        for j in range(rows_per_w):
            copies.append(
                pltpu.async_copy(utab_hbm.at[uidx_v.at[j]], urows_v.at[j], sem))
            copies.append(
                pltpu.async_copy(itab_hbm.at[iidx_v.at[j]], irows_v.at[j], sem))
        for cp in copies:
            cp.wait()
        pltpu.sync_copy(urows_v, uout_hbm.at[pl.ds(base, rows_per_w)])
        pltpu.sync_copy(irows_v, iout_hbm.at[pl.ds(base, rows_per_w)])

    return gather_kernel


def _mlp_body(u_ref, i_ref, w1_ref, b1_ref, w2_ref, out_ref):
    u = u_ref[...]
    it = i_ref[...]
    w1 = w1_ref[...]
    wa = w1[:, :EMB_K]
    wb = w1[:, EMB_K:]
    dn = (((1,), (1,)), ((), ()))
    h = lax.dot_general(u, wa, dn, preferred_element_type=jnp.float32)
    h = h + lax.dot_general(it, wb, dn, preferred_element_type=jnp.float32)
    h = jnp.maximum(h + b1_ref[...], 0.0)
    out_ref[...] = lax.dot_general(h, w2_ref[...], dn,
                                   preferred_element_type=jnp.float32)


def _mlp(u, it, W1, b1, W2, blk):
    batch = u.shape[0]
    grid = (batch // blk,)
    return pl.pallas_call(
        _mlp_body,
        grid=grid,
        in_specs=[
            pl.BlockSpec((blk, EMB_K), lambda b: (b, 0)),
            pl.BlockSpec((blk, EMB_K), lambda b: (b, 0)),
            pl.BlockSpec((EMB_K, 2 * EMB_K), lambda b: (0, 0)),
            pl.BlockSpec((1, EMB_K), lambda b: (0, 0)),
            pl.BlockSpec((1, EMB_K), lambda b: (0, 0)),
        ],
        out_specs=pl.BlockSpec((blk, 1), lambda b: (b, 0)),
        out_shape=jax.ShapeDtypeStruct((batch, 1), jnp.float32),
    )(u, it, W1, b1.reshape(1, EMB_K), W2)


def kernel(x, user_table, item_table, W1, b1, W2):
    batch = x.shape[0]
    emb_k = user_table.shape[1]
    n_workers = 32
    uidx = x[:, 0].reshape(batch // IDX_W, IDX_W)
    iidx = x[:, 1].reshape(batch // IDX_W, IDX_W)
    gk = _make_gather_kernel(batch, emb_k, n_workers)
    u3, i3 = gk(uidx, iidx, user_table, item_table)
    user_embed = u3.reshape(batch, emb_k)
    item_embed = i3.reshape(batch, emb_k)
    out = _mlp(user_embed, item_embed, W1, b1, W2, blk=2048)
    return (out, user_embed, item_embed)


# speedup vs baseline: 1.5615x; 1.5615x over previous
import functools

import jax
import jax.numpy as jnp
from jax import lax
from jax.experimental import pallas as pl
from jax.experimental.pallas import tpu as pltpu
from jax.experimental.pallas import tpu_sc as plsc

EMB_K = 64
N_WORKERS = 32


def _make_gather_kernel(batch, emb_k):
    per_w = batch // N_WORKERS
    ch = per_w // 2
    mesh = plsc.VectorSubcoreMesh(core_axis_name="c", subcore_axis_name="s")

    @functools.partial(
        pl.kernel,
        mesh=mesh,
        compiler_params=pltpu.CompilerParams(needs_layout_passes=False),
        out_type=[
            jax.ShapeDtypeStruct((batch, emb_k), jnp.float32),
            jax.ShapeDtypeStruct((batch, emb_k), jnp.float32),
        ],
        scratch_types=[
            pltpu.VMEM((per_w,), jnp.int32),
            pltpu.VMEM((per_w,), jnp.int32),
            pltpu.VMEM((ch, emb_k), jnp.float32),
            pltpu.VMEM((ch, emb_k), jnp.float32),
            pltpu.SemaphoreType.DMA,
            pltpu.SemaphoreType.DMA,
        ],
    )
    def gather_kernel(uidx_hbm, iidx_hbm, utab_hbm, itab_hbm,
                      uout_hbm, iout_hbm,
                      uidx_v, iidx_v, buf0, buf1, sem0, sem1):
        wid = lax.axis_index("s") * 2 + lax.axis_index("c")
        base = wid * per_w
        pltpu.sync_copy(uidx_hbm.at[pl.ds(base, per_w)], uidx_v)
        pltpu.sync_copy(iidx_hbm.at[pl.ds(base, per_w)], iidx_v)
        lane = lax.iota(jnp.int32, 16)

        def fire(tab, idx_ref, idx_off, buf, sem):
            def group(g, c):
                idxv = idx_ref[pl.ds(idx_off + g * 16, 16)]
                for l in range(16):
                    s = jnp.sum(jnp.where(lane == l, idxv, 0))
                    pltpu.async_copy(tab.at[pl.ds(s, 1)],
                                     buf.at[pl.ds(g * 16 + l, 1)], sem)
                return c
            lax.fori_loop(0, ch // 16, group, 0)

        def drain_store(tab, buf, sem, out, out_off):
            pltpu.make_async_copy(tab.at[pl.ds(0, ch)], buf, sem).wait()
            pltpu.sync_copy(buf, out.at[pl.ds(out_off, ch)])

        batches = [
            (utab_hbm, uidx_v, 0, uout_hbm, base),
            (utab_hbm, uidx_v, ch, uout_hbm, base + ch),
            (itab_hbm, iidx_v, 0, iout_hbm, base),
            (itab_hbm, iidx_v, ch, iout_hbm, base + ch),
        ]
        bufs = (buf0, buf1)
        sems = (sem0, sem1)
        fire(batches[0][0], batches[0][1], batches[0][2], bufs[0], sems[0])
        fire(batches[1][0], batches[1][1], batches[1][2], bufs[1], sems[1])
        for b in range(2, 4):
            tab, idx_ref, idx_off, out, out_off = batches[b - 2]
            drain_store(tab, bufs[b % 2], sems[b % 2], out, out_off)
            tab, idx_ref, idx_off, out, out_off = batches[b]
            fire(tab, idx_ref, idx_off, bufs[b % 2], sems[b % 2])
        for b in range(2, 4):
            tab, idx_ref, idx_off, out, out_off = batches[b]
            drain_store(tab, bufs[b % 2], sems[b % 2], out, out_off)

    return gather_kernel


def _mlp_body(u_ref, i_ref, w1_ref, b1_ref, w2_ref, out_ref):
    u = u_ref[...]
    it = i_ref[...]
    w1 = w1_ref[...]
    wa = w1[:, :EMB_K]
    wb = w1[:, EMB_K:]
    dn = (((1,), (1,)), ((), ()))
    h = lax.dot_general(u, wa, dn, preferred_element_type=jnp.float32)
    h = h + lax.dot_general(it, wb, dn, preferred_element_type=jnp.float32)
    h = jnp.maximum(h + b1_ref[...], 0.0)
    out_ref[...] = lax.dot_general(h, w2_ref[...], dn,
                                   preferred_element_type=jnp.float32)


def _mlp(u, it, W1, b1, W2, blk):
    batch = u.shape[0]
    grid = (batch // blk,)
    return pl.pallas_call(
        _mlp_body,
        grid=grid,
        in_specs=[
            pl.BlockSpec((blk, EMB_K), lambda b: (b, 0)),
            pl.BlockSpec((blk, EMB_K), lambda b: (b, 0)),
            pl.BlockSpec((EMB_K, 2 * EMB_K), lambda b: (0, 0)),
            pl.BlockSpec((1, EMB_K), lambda b: (0, 0)),
            pl.BlockSpec((1, EMB_K), lambda b: (0, 0)),
        ],
        out_specs=pl.BlockSpec((blk, 1), lambda b: (b, 0)),
        out_shape=jax.ShapeDtypeStruct((batch, 1), jnp.float32),
    )(u, it, W1, b1.reshape(1, EMB_K), W2)


def kernel(x, user_table, item_table, W1, b1, W2):
    batch = x.shape[0]
    emb_k = user_table.shape[1]
    uidx = x[:, 0]
    iidx = x[:, 1]
    gk = _make_gather_kernel(batch, emb_k)
    user_embed, item_embed = gk(uidx, iidx, user_table, item_table)
    out = _mlp(user_embed, item_embed, W1, b1, W2, blk=2048)
    return (out, user_embed, item_embed)
